# baseline (speedup 1.0000x reference)
import functools

import jax
import jax.numpy as jnp
from jax import lax
from jax.experimental import pallas as pl
from jax.experimental.pallas import tpu as pltpu
from jax.experimental.pallas import tpu_sc as plsc

SHORT_LEN = 131072
MEDIUM_LEN = 32768
LONG_LEN = 8192
EMBED_DIM = 256
B = 16384
TOTAL = SHORT_LEN + MEDIUM_LEN + LONG_LEN

NW = 32
RPW = SHORT_LEN // NW
MEDPW = MEDIUM_LEN // NW
LONGPW = LONG_LEN // NW
L = 16
NV = B // L
CH = 128
NCH = B // CH
WN = RPW + L


def _hm_body(short_hbm, med_hbm, long_hbm, upd_hbm, idx_hbm, out_hbm,
             idxbuf, winner, rowlist, jlist, rowbuf, rowbuf2,
             sem_ia, sem_ib, sem_oa, sem_ob, sem_g, sem_i):
    w = lax.axis_index("s") * 2 + lax.axis_index("c")
    base = w * RPW

    cp_idx = pltpu.make_async_copy(idx_hbm, idxbuf, sem_i)
    cp_idx.start()

    lanes = lax.iota(jnp.int32, L)
    neg1 = jnp.full((L,), -1, jnp.int32)

    def init_body(i, _):
        winner[pl.ds(pl.multiple_of(i * L, L), L)] = neg1
        return 0
    lax.fori_loop(0, WN // L, init_body, 0)

    cp_idx.wait()

    def dedup_step(i):
        v = idxbuf[pl.ds(pl.multiple_of(i * L, L), L)]
        rloc = v - base
        inb = (rloc >= 0) & (rloc < RPW)
        x = jnp.where(inb, rloc, RPW)
        _, last = plsc.scan_count(x)
        jvec = i * L + lanes
        plsc.store_scatter(winner, [x], jvec, mask=last)

    def ring_copy(src, src_off, dst_off, nchunks, work, steps, carry):
        def cin(c, buf, sem):
            return pltpu.make_async_copy(
                src.at[pl.ds(src_off + c * CH, CH)], buf, sem)

        def cout(c, buf, sem):
            return pltpu.make_async_copy(
                buf, out_hbm.at[pl.ds(dst_off + c * CH, CH)], sem)

        def do_work(t0, carry):
            def wl(k, c):
                return work(t0 + k, c)
            return lax.fori_loop(0, steps, wl, carry) if steps else carry

        cin(0, rowbuf, sem_ia).start()
        cin(1, rowbuf2, sem_ib).start()
        carry = do_work(0, carry)
        cin(0, rowbuf, sem_ia).wait()
        cout(0, rowbuf, sem_oa).start()
        carry = do_work(steps, carry)
        cin(1, rowbuf2, sem_ib).wait()
        cout(1, rowbuf2, sem_ob).start()

        def body(p, carry):
            ca, cb = 2 * p, 2 * p + 1
            cout(ca - 2, rowbuf, sem_oa).wait()
            cin(ca, rowbuf, sem_ia).start()
            carry = do_work(2 * p * steps, carry)
            cout(cb - 2, rowbuf2, sem_ob).wait()
            cin(cb, rowbuf2, sem_ib).start()
            carry = do_work((2 * p + 1) * steps, carry)
            cin(ca, rowbuf, sem_ia).wait()
            cout(ca, rowbuf, sem_oa).start()
            cin(cb, rowbuf2, sem_ib).wait()
            cout(cb, rowbuf2, sem_ob).start()
            return carry
        carry = lax.fori_loop(1, nchunks // 2, body, carry)

        cout(nchunks - 2, rowbuf, sem_oa).wait()
        cout(nchunks - 1, rowbuf2, sem_ob).wait()
        return carry

    def dedup_work(t, c):
        dedup_step(t)
        return c
    ring_copy(short_hbm, base, base, RPW // CH, dedup_work, NV // (RPW // CH),
              jnp.int32(0))

    def compact_step(i, carry):
        cnt, lastpair = carry
        wv = winner[pl.ds(pl.multiple_of(i * L, L), L)]
        m = wv >= 0
        mi = m.astype(jnp.int32)
        pos = cnt + plsc.cumsum(mi) - 1
        rowg = base + i * L + lanes
        plsc.store_scatter(rowlist, [pos >> 7, pos & 127], rowg, mask=m)
        plsc.store_scatter(jlist, [pos >> 7, pos & 127], wv, mask=m)
        pair = jnp.where(m, ((i * L + lanes) << 14) | wv, -1)
        return cnt + jnp.sum(mi), jnp.maximum(lastpair, jnp.max(pair))

    cnt, lastpair = ring_copy(
        med_hbm, w * MEDPW, SHORT_LEN + w * MEDPW, MEDPW // CH,
        compact_step, (RPW // L) // (MEDPW // CH), (jnp.int32(0), jnp.int32(-1)))

    ring_copy(long_hbm, w * LONGPW, SHORT_LEN + MEDIUM_LEN + w * LONGPW,
              LONGPW // CH, None, 0, jnp.int32(0))

    nch2 = (cnt + 2 * CH - 1) >> 8
    padded = nch2 * 2 * CH
    padrow = jnp.full((L,), base, jnp.int32) + (lastpair >> 14)
    padj = jnp.full((L,), 0, jnp.int32) + (lastpair & (B - 1))

    def pad_body(q, _):
        p = q * L + lanes
        m = (p >= cnt) & (p < padded)
        plsc.store_scatter(rowlist, [p >> 7, p & 127], padrow, mask=m)
        plsc.store_scatter(jlist, [p >> 7, p & 127], padj, mask=m)
        return 0
    lax.fori_loop(cnt >> 4, (padded + L - 1) >> 4, pad_body, 0)

    def gat(c, buf, sem):
        return pltpu.make_async_copy(upd_hbm.at[jlist.at[c]], buf, sem)

    def sct(c, buf, sem):
        return pltpu.make_async_copy(buf, out_hbm.at[rowlist.at[c]], sem)

    def pair_body(q, _):
        @pl.when(q > 0)
        def _():
            sct(2 * q - 2, rowbuf, sem_oa).wait()
        ga = gat(2 * q, rowbuf, sem_ia)
        ga.start()

        @pl.when(q > 0)
        def _():
            sct(2 * q - 1, rowbuf2, sem_ob).wait()
        gb = gat(2 * q + 1, rowbuf2, sem_ib)
        gb.start()
        ga.wait()
        sct(2 * q, rowbuf, sem_oa).start()
        gb.wait()
        sct(2 * q + 1, rowbuf2, sem_ob).start()
        return 0
    lax.fori_loop(0, nch2, pair_body, 0)

    @pl.when(nch2 > 0)
    def _():
        sct(2 * nch2 - 2, rowbuf, sem_oa).wait()
        sct(2 * nch2 - 1, rowbuf2, sem_ob).wait()


_hm_kernel = functools.partial(
    pl.kernel,
    out_type=jax.ShapeDtypeStruct((TOTAL, EMBED_DIM), jnp.float32),
    mesh=plsc.VectorSubcoreMesh(core_axis_name="c", subcore_axis_name="s"),
    compiler_params=pltpu.CompilerParams(needs_layout_passes=False),
    scratch_types=[
        pltpu.VMEM((B,), jnp.int32),
        pltpu.VMEM((WN,), jnp.int32),
        pltpu.VMEM((NCH, CH), jnp.int32),
        pltpu.VMEM((NCH, CH), jnp.int32),
        pltpu.VMEM((CH, EMBED_DIM), jnp.float32),
        pltpu.VMEM((CH, EMBED_DIM), jnp.float32),
        pltpu.SemaphoreType.DMA,
        pltpu.SemaphoreType.DMA,
        pltpu.SemaphoreType.DMA,
        pltpu.SemaphoreType.DMA,
        pltpu.SemaphoreType.DMA,
        pltpu.SemaphoreType.DMA,
    ],
)(_hm_body)


@jax.jit
def kernel(short_mem, medium_mem, long_mem, updates, short_idx):
    return _hm_kernel(short_mem, medium_mem, long_mem, updates,
                      short_idx.astype(jnp.int32))
